# baseline (speedup 1.0000x reference)
import functools

import jax
import jax.numpy as jnp
from jax import lax
from jax.experimental import pallas as pl
from jax.experimental.pallas import tpu as pltpu
from jax.experimental.pallas import tpu_sc as plsc

B = 16384
NC, NS = 2, 16
NW = NC * NS
BPW = B // NW
CHUNK = 128
NCHUNK = BPW // CHUNK
L = 16


def _sc_gather_body(uids, iids, gut, git, mut, mit,
                    gu_o, gi_o, mu_o, mi_o,
                    uix, iix, gux, gix, mux, mix,
                    guv, giv, muv, miv, sem):
    wid = lax.axis_index("s") * NC + lax.axis_index("c")
    base = wid * BPW
    for c in range(NCHUNK):
        pltpu.sync_copy(uids.at[pl.ds(base + c * CHUNK, CHUNK)], uix.at[c])
        pltpu.sync_copy(iids.at[pl.ds(base + c * CHUNK, CHUNK)], iix.at[c])
    for c in range(NCHUNK):
        for k in range(CHUNK // L):
            sl = pl.ds(k * L, L)
            u = uix[c, sl]
            i = iix[c, sl]
            gux[c, sl] = lax.shift_right_logical(u, 2)
            mux[c, sl] = lax.shift_right_logical(u, 3)
            gix[c, sl] = lax.shift_right_logical(i, 2)
            mix[c, sl] = lax.shift_right_logical(i, 3)
    for c in range(NCHUNK):
        copies = [
            pltpu.async_copy(gut.at[gux.at[c]], guv, sem),
            pltpu.async_copy(git.at[gix.at[c]], giv, sem),
            pltpu.async_copy(mut.at[mux.at[c]], muv, sem),
            pltpu.async_copy(mit.at[mix.at[c]], miv, sem),
        ]
        for cp in copies:
            cp.wait()
        osl = pl.ds(base + c * CHUNK, CHUNK)
        pltpu.sync_copy(guv, gu_o.at[osl])
        pltpu.sync_copy(giv, gi_o.at[osl])
        pltpu.sync_copy(muv, mu_o.at[osl])
        pltpu.sync_copy(miv, mi_o.at[osl])


_sc_gather = functools.partial(
    pl.kernel,
    out_type=(
        jax.ShapeDtypeStruct((B, 128), jnp.float32),
        jax.ShapeDtypeStruct((B, 128), jnp.float32),
        jax.ShapeDtypeStruct((B, 128), jnp.float32),
        jax.ShapeDtypeStruct((B, 128), jnp.float32),
    ),
    mesh=plsc.VectorSubcoreMesh(core_axis_name="c", subcore_axis_name="s"),
    scratch_types=[
        pltpu.VMEM((NCHUNK, CHUNK), jnp.int32),
        pltpu.VMEM((NCHUNK, CHUNK), jnp.int32),
        pltpu.VMEM((NCHUNK, CHUNK), jnp.int32),
        pltpu.VMEM((NCHUNK, CHUNK), jnp.int32),
        pltpu.VMEM((NCHUNK, CHUNK), jnp.int32),
        pltpu.VMEM((NCHUNK, CHUNK), jnp.int32),
        pltpu.VMEM((CHUNK, 128), jnp.float32),
        pltpu.VMEM((CHUNK, 128), jnp.float32),
        pltpu.VMEM((CHUNK, 128), jnp.float32),
        pltpu.VMEM((CHUNK, 128), jnp.float32),
        pltpu.SemaphoreType.DMA,
    ],
)(_sc_gather_body)


def _dense_body(uq, iq, gup, gip, mup, mip,
                w1a, w1b, b1, w2, b2, w3, b3, wpg, wph, bp, out):
    uqv = uq[...]
    iqv = iq[...]
    gu = jnp.zeros(gup.shape[:1] + (32,), jnp.float32)
    gi = jnp.zeros_like(gu)
    for q in range(4):
        sl = slice(q * 32, (q + 1) * 32)
        gu += jnp.where((uqv & 3) == q, 1.0, 0.0) * gup[:, sl]
        gi += jnp.where((iqv & 3) == q, 1.0, 0.0) * gip[:, sl]
    mu = jnp.zeros(gup.shape[:1] + (16,), jnp.float32)
    mi = jnp.zeros_like(mu)
    for q in range(8):
        sl = slice(q * 16, (q + 1) * 16)
        mu += jnp.where(uqv == q, 1.0, 0.0) * mup[:, sl]
        mi += jnp.where(iqv == q, 1.0, 0.0) * mip[:, sl]
    h = jnp.dot(mu, w1a[...], preferred_element_type=jnp.float32)
    h += jnp.dot(mi, w1b[...], preferred_element_type=jnp.float32)
    h = jnp.maximum(h + b1[...], 0.0)
    h = jnp.maximum(jnp.dot(h, w2[...], preferred_element_type=jnp.float32) + b2[...], 0.0)
    h = jnp.maximum(jnp.dot(h, w3[...], preferred_element_type=jnp.float32) + b3[...], 0.0)
    g = gu * gi
    logit = jnp.sum(g * wpg[...], axis=1, keepdims=True)
    logit += jnp.sum(h * wph[...], axis=1, keepdims=True)
    logit += bp[...]
    out[...] = 1.0 / (1.0 + jnp.exp(-logit))


def kernel(user_ids, item_ids, gmf_user_table, gmf_item_table,
           mlp_user_table, mlp_item_table, W1, b1, W2, b2, W3, b3, Wp, bp):
    n_users = gmf_user_table.shape[0]
    n_items = gmf_item_table.shape[0]
    gup, gip, mup, mip = _sc_gather(
        user_ids.astype(jnp.int32), item_ids.astype(jnp.int32),
        gmf_user_table.reshape(n_users // 4, 128),
        gmf_item_table.reshape(n_items // 4, 128),
        mlp_user_table.reshape(n_users // 8, 128),
        mlp_item_table.reshape(n_items // 8, 128),
    )
    uq = (user_ids & 7).astype(jnp.int32).reshape(B, 1)
    iq = (item_ids & 7).astype(jnp.int32).reshape(B, 1)
    w1a, w1b = W1[:16, :], W1[16:, :]
    wpg = Wp[:32, 0].reshape(1, 32)
    wph = Wp[32:, 0].reshape(1, 8)
    BLK = 2048
    out = pl.pallas_call(
        _dense_body,
        grid=(B // BLK,),
        in_specs=[
            pl.BlockSpec((BLK, 1), lambda i: (i, 0)),
            pl.BlockSpec((BLK, 1), lambda i: (i, 0)),
            pl.BlockSpec((BLK, 128), lambda i: (i, 0)),
            pl.BlockSpec((BLK, 128), lambda i: (i, 0)),
            pl.BlockSpec((BLK, 128), lambda i: (i, 0)),
            pl.BlockSpec((BLK, 128), lambda i: (i, 0)),
            pl.BlockSpec((16, 32), lambda i: (0, 0)),
            pl.BlockSpec((16, 32), lambda i: (0, 0)),
            pl.BlockSpec((1, 32), lambda i: (0, 0)),
            pl.BlockSpec((32, 16), lambda i: (0, 0)),
            pl.BlockSpec((1, 16), lambda i: (0, 0)),
            pl.BlockSpec((16, 8), lambda i: (0, 0)),
            pl.BlockSpec((1, 8), lambda i: (0, 0)),
            pl.BlockSpec((1, 32), lambda i: (0, 0)),
            pl.BlockSpec((1, 8), lambda i: (0, 0)),
            pl.BlockSpec((1, 1), lambda i: (0, 0)),
        ],
        out_specs=pl.BlockSpec((BLK, 1), lambda i: (i, 0)),
        out_shape=jax.ShapeDtypeStruct((B, 1), jnp.float32),
    )(uq, iq, gup, gip, mup, mip, w1a, w1b, b1.reshape(1, 32), W2,
      b2.reshape(1, 16), W3, b3.reshape(1, 8), wpg, wph, bp.reshape(1, 1))
    return out
